# baseline (speedup 1.0000x reference)
import functools

import jax
import jax.numpy as jnp
from jax import lax
from jax.experimental import pallas as pl
from jax.experimental.pallas import tpu as pltpu
from jax.experimental.pallas import tpu_sc as plsc

NC = 2
NS = 16
L = 16
NW = NC * NS

_N = 10000
_E = 320000
_D = 128
_H = 64
_G = 512

NP = 10240
K = 128
NCH = 80
EP = NW * NCH * K
NBUF = 4
RNDS = NCH // NBUF
RPT = NP // NS
GS = _G // NS

_mesh = plsc.VectorSubcoreMesh(
    core_axis_name="c", subcore_axis_name="s", num_cores=NC, num_subcores=NS
)

_SC_PARAMS = pltpu.CompilerParams(
    needs_layout_passes=False, use_tc_tiling_on_sc=False
)

@functools.partial(
    pl.kernel,
    out_type=jax.ShapeDtypeStruct((NC, NP), jnp.float32),
    mesh=_mesh,
    compiler_params=_SC_PARAMS,
    scratch_types=[
        pltpu.VMEM((NCH, K), jnp.int32),
        pltpu.VMEM((NP,), jnp.float32),
        pltpu.VMEM((NS, RPT), jnp.float32),
        pltpu.VMEM((RPT,), jnp.float32),
        pltpu.VMEM_SHARED((NS, NP), jnp.float32),
    ],
)
def _deg_kernel(col_hbm, deg_out, colv, degp, comb, outv, dsh):
    c = lax.axis_index("c")
    sid = lax.axis_index("s")
    wid = c * NS + sid

    zero16 = jnp.zeros((L,), jnp.float32)

    @pl.loop(0, NP // L)
    def _(i):
        degp[pl.ds(i * L, L)] = zero16

    pltpu.sync_copy(col_hbm.at[wid], colv)
    ones = jnp.ones((L,), jnp.float32)

    @pl.loop(0, NCH)
    def _(j):
        for t in range(K // L):
            idx = colv[j, pl.ds(t * L, L)]
            plsc.addupdate_scatter(degp, [idx], ones)

    pltpu.sync_copy(degp, dsh.at[sid])
    plsc.subcore_barrier()
    pltpu.sync_copy(dsh.at[:, pl.ds(sid * RPT, RPT)], comb)

    @pl.loop(0, RPT // L)
    def _(t):
        a = comb[0, pl.ds(t * L, L)]
        for r in range(1, NS):
            a = a + comb[r, pl.ds(t * L, L)]
        outv[pl.ds(t * L, L)] = a

    pltpu.sync_copy(outv, deg_out.at[c, pl.ds(sid * RPT, RPT)])


@functools.partial(
    pl.kernel,
    out_type=jax.ShapeDtypeStruct((NC, NP, _H), jnp.float32),
    mesh=_mesh,
    compiler_params=_SC_PARAMS,
    scratch_types=[
        pltpu.VMEM((NCH, K), jnp.int32),
        pltpu.VMEM((NCH, K), jnp.int32),
        pltpu.VMEM((NBUF, K, _H), jnp.float32),
        pltpu.SemaphoreType.DMA((NBUF,)),
        pltpu.SemaphoreType.DMA((NBUF,)),
        pltpu.VMEM_SHARED((NP, _H), jnp.float32),
    ],
)
def _agg_kernel(y_hbm, row_hbm, col_hbm, zero_hbm, acc_out, rowv, colv, msg,
                semg, sems, acc_sh):
    c = lax.axis_index("c")
    sid = lax.axis_index("s")
    wid = c * NS + sid

    pltpu.sync_copy(zero_hbm.at[pl.ds(sid * RPT, RPT)], acc_sh.at[pl.ds(sid * RPT, RPT)])
    pltpu.sync_copy(row_hbm.at[wid], rowv)
    pltpu.sync_copy(col_hbm.at[wid], colv)
    plsc.subcore_barrier()

    pltpu.async_copy(y_hbm.at[rowv.at[0]], msg.at[0], semg.at[0])

    @pl.loop(0, RNDS)
    def _(g):
        for b in range(NBUF):
            j = g * NBUF + b
            b1 = (b + 1) % NBUF

            def _free():
                pltpu.make_async_copy(
                    msg.at[b1], acc_sh.at[colv.at[j - (NBUF - 1)]], sems.at[b1]
                ).wait()

            if b == NBUF - 1:
                _free()
            else:
                pl.when(g > 0)(_free)

            def _pref():
                pltpu.async_copy(y_hbm.at[rowv.at[j + 1]], msg.at[b1], semg.at[b1])

            if b == NBUF - 1:
                pl.when(g < RNDS - 1)(_pref)
            else:
                _pref()

            pltpu.make_async_copy(y_hbm.at[rowv.at[j]], msg.at[b], semg.at[b]).wait()
            pltpu.async_copy(msg.at[b], acc_sh.at[colv.at[j]], sems.at[b], add=True)

    for b in range(1, NBUF):
        j = NCH - NBUF + b
        pltpu.make_async_copy(
            msg.at[b], acc_sh.at[colv.at[j]], sems.at[b]
        ).wait()

    plsc.subcore_barrier()
    pltpu.sync_copy(
        acc_sh.at[pl.ds(sid * RPT, RPT)], acc_out.at[c, pl.ds(sid * RPT, RPT)]
    )


BLK = 1024


def _dinv_of(deg_ref):
    deg = deg_ref[0, :] + deg_ref[1, :] + 1.0
    return 1.0 / jnp.sqrt(deg)


def _mm0_body(x_ref, w_ref, xw_ref):
    xw_ref[...] = jnp.dot(x_ref[...], w_ref[...],
                          preferred_element_type=jnp.float32,
                          precision=lax.Precision.HIGHEST)


def _mm1_body(xw_ref, deg_ref, y_ref):
    dinv = _dinv_of(deg_ref)
    y_ref[...] = xw_ref[...] * dinv[:, None]


def _mm2_body(acc_ref, y1_ref, deg_ref, b1_ref, w2_ref, y2_ref):
    dinv = _dinv_of(deg_ref)
    a = (acc_ref[0] + acc_ref[1] + y1_ref[...]) * dinv[:, None] + b1_ref[...]
    h = jnp.maximum(a, 0.0)
    y2_ref[...] = (
        jnp.dot(h, w2_ref[...], preferred_element_type=jnp.float32,
                precision=lax.Precision.HIGHEST) * dinv[:, None]
    )


def _mm3_body(acc_ref, y2_ref, deg_ref, b2_ref, fcw_ref, batch_ref, fcb_ref,
              out_ref, sums_ref, cnt_ref):
    i = pl.program_id(0)
    dinv = _dinv_of(deg_ref)
    h = (acc_ref[0] + acc_ref[1] + y2_ref[...]) * dinv[:, None] + b2_ref[...]
    s = jnp.dot(h, fcw_ref[...], preferred_element_type=jnp.float32,
                precision=lax.Precision.HIGHEST)
    rowid = i * BLK + lax.broadcasted_iota(jnp.int32, (BLK, 1), 0)
    gids = lax.broadcasted_iota(jnp.int32, (1, _G), 1)
    onehot = jnp.where((batch_ref[...] == gids) & (rowid < _N), 1.0, 0.0)
    ps = jnp.sum(onehot * s, axis=0)[None, :]
    pc = jnp.sum(onehot, axis=0)[None, :]

    @pl.when(i == 0)
    def _():
        sums_ref[...] = ps
        cnt_ref[...] = pc

    @pl.when(i > 0)
    def _():
        sums_ref[...] += ps
        cnt_ref[...] += pc

    out_ref[...] = sums_ref[...] / jnp.maximum(cnt_ref[...], 1.0) + fcb_ref[...]


_GRID = NP // BLK

_mm0 = pl.pallas_call(
    _mm0_body,
    grid=(_GRID,),
    in_specs=[
        pl.BlockSpec((BLK, _D), lambda i: (i, 0)),
        pl.BlockSpec((_D, _H), lambda i: (0, 0)),
    ],
    out_specs=pl.BlockSpec((BLK, _H), lambda i: (i, 0)),
    out_shape=jax.ShapeDtypeStruct((NP, _H), jnp.float32),
)

_mm1 = pl.pallas_call(
    _mm1_body,
    grid=(_GRID,),
    in_specs=[
        pl.BlockSpec((BLK, _H), lambda i: (i, 0)),
        pl.BlockSpec((NC, BLK), lambda i: (0, i)),
    ],
    out_specs=pl.BlockSpec((BLK, _H), lambda i: (i, 0)),
    out_shape=jax.ShapeDtypeStruct((NP, _H), jnp.float32),
)

_mm2 = pl.pallas_call(
    _mm2_body,
    grid=(_GRID,),
    in_specs=[
        pl.BlockSpec((NC, BLK, _H), lambda i: (0, i, 0)),
        pl.BlockSpec((BLK, _H), lambda i: (i, 0)),
        pl.BlockSpec((NC, BLK), lambda i: (0, i)),
        pl.BlockSpec((1, _H), lambda i: (0, 0)),
        pl.BlockSpec((_H, _H), lambda i: (0, 0)),
    ],
    out_specs=pl.BlockSpec((BLK, _H), lambda i: (i, 0)),
    out_shape=jax.ShapeDtypeStruct((NP, _H), jnp.float32),
)

_mm3 = pl.pallas_call(
    _mm3_body,
    grid=(_GRID,),
    in_specs=[
        pl.BlockSpec((NC, BLK, _H), lambda i: (0, i, 0)),
        pl.BlockSpec((BLK, _H), lambda i: (i, 0)),
        pl.BlockSpec((NC, BLK), lambda i: (0, i)),
        pl.BlockSpec((1, _H), lambda i: (0, 0)),
        pl.BlockSpec((_H, 1), lambda i: (0, 0)),
        pl.BlockSpec((BLK, 1), lambda i: (i, 0)),
        pl.BlockSpec((1, 1), lambda i: (0, 0)),
    ],
    out_specs=pl.BlockSpec((1, _G), lambda i: (0, 0)),
    out_shape=jax.ShapeDtypeStruct((1, _G), jnp.float32),
    scratch_shapes=[
        pltpu.VMEM((1, _G), jnp.float32),
        pltpu.VMEM((1, _G), jnp.float32),
    ],
)


def kernel(x, edge_index, batch, W1, b1, W2, b2, fcW, fcb):
    x_pad = jnp.pad(x, ((0, NP - _N), (0, 0)))
    pad_ids = _N + jnp.arange(EP - _E, dtype=jnp.int32) % (NP - _N)
    epad = jnp.stack([pad_ids, pad_ids])
    eidx = jnp.concatenate([edge_index, epad], axis=1)
    row_r = eidx[0].reshape(NW, NCH, K)
    col_r = eidx[1].reshape(NW, NCH, K)
    batch2d = jnp.pad(batch, (0, NP - _N)).reshape(NP, 1)
    zeros_nh = jnp.zeros((NP, _H), jnp.float32)
    b1r = b1.reshape(1, _H)
    b2r = b2.reshape(1, _H)

    xw1 = _mm0(x_pad, W1)
    deg2 = _deg_kernel(col_r)
    y1 = _mm1(xw1, deg2)
    acc1 = _agg_kernel(y1, row_r, col_r, zeros_nh)
    y2 = _mm2(acc1, y1, deg2, b1r, W2)
    acc2 = _agg_kernel(y2, row_r, col_r, zeros_nh)
    pooled = _mm3(acc2, y2, deg2, b2r, fcW, batch2d, fcb.reshape(1, 1))
    return pooled.reshape(_G, 1)


# speedup vs baseline: 1.0716x; 1.0716x over previous
import functools

import jax
import jax.numpy as jnp
from jax import lax
from jax.experimental import pallas as pl
from jax.experimental.pallas import tpu as pltpu
from jax.experimental.pallas import tpu_sc as plsc

NC = 2
NS = 16
L = 16
NW = NC * NS

_N = 10000
_E = 320000
_D = 128
_H = 64
_G = 512

NP = 10240
K = 128
NCH = 80
EP = NW * NCH * K
NBUF = 4
GLEAD = 2
RNDS = NCH // NBUF
RPT = NP // NS
GS = _G // NS

_mesh = plsc.VectorSubcoreMesh(
    core_axis_name="c", subcore_axis_name="s", num_cores=NC, num_subcores=NS
)

_SC_PARAMS = pltpu.CompilerParams(
    needs_layout_passes=False, use_tc_tiling_on_sc=False
)

@functools.partial(
    pl.kernel,
    out_type=jax.ShapeDtypeStruct((NC, NP), jnp.float32),
    mesh=_mesh,
    compiler_params=_SC_PARAMS,
    scratch_types=[
        pltpu.VMEM((NCH, K), jnp.int32),
        pltpu.VMEM((NP,), jnp.float32),
        pltpu.VMEM((NS, RPT), jnp.float32),
        pltpu.VMEM((RPT,), jnp.float32),
        pltpu.VMEM_SHARED((NS, NP), jnp.float32),
    ],
)
def _deg_kernel(col_hbm, deg_out, colv, degp, comb, outv, dsh):
    c = lax.axis_index("c")
    sid = lax.axis_index("s")
    wid = c * NS + sid

    zero16 = jnp.zeros((L,), jnp.float32)

    @pl.loop(0, NP // L)
    def _(i):
        degp[pl.ds(i * L, L)] = zero16

    pltpu.sync_copy(col_hbm.at[wid], colv)
    ones = jnp.ones((L,), jnp.float32)

    @pl.loop(0, NCH)
    def _(j):
        for t in range(K // L):
            idx = colv[j, pl.ds(t * L, L)]
            plsc.addupdate_scatter(degp, [idx], ones)

    pltpu.sync_copy(degp, dsh.at[sid])
    plsc.subcore_barrier()
    pltpu.sync_copy(dsh.at[:, pl.ds(sid * RPT, RPT)], comb)

    @pl.loop(0, RPT // L)
    def _(t):
        a = comb[0, pl.ds(t * L, L)]
        for r in range(1, NS):
            a = a + comb[r, pl.ds(t * L, L)]
        outv[pl.ds(t * L, L)] = a

    pltpu.sync_copy(outv, deg_out.at[c, pl.ds(sid * RPT, RPT)])


@functools.partial(
    pl.kernel,
    out_type=jax.ShapeDtypeStruct((NC, NP, _H), jnp.float32),
    mesh=_mesh,
    compiler_params=_SC_PARAMS,
    scratch_types=[
        pltpu.VMEM((NCH, K), jnp.int32),
        pltpu.VMEM((NCH, K), jnp.int32),
        pltpu.VMEM((NBUF, K, _H), jnp.float32),
        pltpu.SemaphoreType.DMA((NBUF,)),
        pltpu.SemaphoreType.DMA((NBUF,)),
        pltpu.VMEM_SHARED((NP, _H), jnp.float32),
    ],
)
def _agg_kernel(y_hbm, row_hbm, col_hbm, zero_hbm, acc_out, rowv, colv, msg,
                semg, sems, acc_sh):
    c = lax.axis_index("c")
    sid = lax.axis_index("s")
    wid = c * NS + sid

    pltpu.sync_copy(zero_hbm.at[pl.ds(sid * RPT, RPT)], acc_sh.at[pl.ds(sid * RPT, RPT)])
    pltpu.sync_copy(row_hbm.at[wid], rowv)
    pltpu.sync_copy(col_hbm.at[wid], colv)
    plsc.subcore_barrier()

    for b0 in range(GLEAD):
        pltpu.async_copy(y_hbm.at[rowv.at[b0]], msg.at[b0], semg.at[b0])

    @pl.loop(0, RNDS)
    def _(g):
        for b in range(NBUF):
            j = g * NBUF + b
            bg = (b + GLEAD) % NBUF

            def _pref():
                def _free():
                    pltpu.make_async_copy(
                        msg.at[bg], acc_sh.at[colv.at[j - (NBUF - GLEAD)]],
                        sems.at[bg]
                    ).wait()

                if b >= NBUF - GLEAD:
                    _free()
                else:
                    pl.when(g > 0)(_free)
                pltpu.async_copy(y_hbm.at[rowv.at[j + GLEAD]], msg.at[bg], semg.at[bg])

            if b >= NBUF - GLEAD:
                pl.when(g < RNDS - 1)(_pref)
            else:
                _pref()

            pltpu.make_async_copy(y_hbm.at[rowv.at[j]], msg.at[b], semg.at[b]).wait()
            pltpu.async_copy(msg.at[b], acc_sh.at[colv.at[j]], sems.at[b], add=True)

    for i in range(NBUF):
        j = NCH - NBUF + i
        pltpu.make_async_copy(
            msg.at[j % NBUF], acc_sh.at[colv.at[j]], sems.at[j % NBUF]
        ).wait()

    plsc.subcore_barrier()
    pltpu.sync_copy(
        acc_sh.at[pl.ds(sid * RPT, RPT)], acc_out.at[c, pl.ds(sid * RPT, RPT)]
    )


BLK = 1024


def _dinv_of(deg_ref):
    deg = deg_ref[0, :] + deg_ref[1, :] + 1.0
    return 1.0 / jnp.sqrt(deg)


def _mm1_body(x_ref, w_ref, deg_ref, y_ref):
    dinv = _dinv_of(deg_ref)
    y_ref[...] = (
        jnp.dot(x_ref[...], w_ref[...], preferred_element_type=jnp.float32,
                precision=lax.Precision.HIGHEST)
        * dinv[:, None]
    )


def _mm2_body(acc_ref, y1_ref, deg_ref, b1_ref, w2_ref, y2_ref):
    dinv = _dinv_of(deg_ref)
    a = (acc_ref[0] + acc_ref[1] + y1_ref[...]) * dinv[:, None] + b1_ref[...]
    h = jnp.maximum(a, 0.0)
    y2_ref[...] = (
        jnp.dot(h, w2_ref[...], preferred_element_type=jnp.float32,
                precision=lax.Precision.HIGHEST) * dinv[:, None]
    )


def _mm3_body(acc_ref, y2_ref, deg_ref, b2_ref, fcw_ref, batch_ref, fcb_ref,
              out_ref, sums_ref, cnt_ref):
    i = pl.program_id(0)
    dinv = _dinv_of(deg_ref)
    h = (acc_ref[0] + acc_ref[1] + y2_ref[...]) * dinv[:, None] + b2_ref[...]
    s = jnp.dot(h, fcw_ref[...], preferred_element_type=jnp.float32,
                precision=lax.Precision.HIGHEST)
    rowid = i * BLK + lax.broadcasted_iota(jnp.int32, (BLK, 1), 0)
    gids = lax.broadcasted_iota(jnp.int32, (1, _G), 1)
    onehot = jnp.where((batch_ref[...] == gids) & (rowid < _N), 1.0, 0.0)
    ps = jnp.sum(onehot * s, axis=0)[None, :]
    pc = jnp.sum(onehot, axis=0)[None, :]

    @pl.when(i == 0)
    def _():
        sums_ref[...] = ps
        cnt_ref[...] = pc

    @pl.when(i > 0)
    def _():
        sums_ref[...] += ps
        cnt_ref[...] += pc

    out_ref[...] = sums_ref[...] / jnp.maximum(cnt_ref[...], 1.0) + fcb_ref[...]


_GRID = NP // BLK

_mm1 = pl.pallas_call(
    _mm1_body,
    grid=(_GRID,),
    in_specs=[
        pl.BlockSpec((BLK, _D), lambda i: (i, 0)),
        pl.BlockSpec((_D, _H), lambda i: (0, 0)),
        pl.BlockSpec((NC, BLK), lambda i: (0, i)),
    ],
    out_specs=pl.BlockSpec((BLK, _H), lambda i: (i, 0)),
    out_shape=jax.ShapeDtypeStruct((NP, _H), jnp.float32),
)

_mm2 = pl.pallas_call(
    _mm2_body,
    grid=(_GRID,),
    in_specs=[
        pl.BlockSpec((NC, BLK, _H), lambda i: (0, i, 0)),
        pl.BlockSpec((BLK, _H), lambda i: (i, 0)),
        pl.BlockSpec((NC, BLK), lambda i: (0, i)),
        pl.BlockSpec((1, _H), lambda i: (0, 0)),
        pl.BlockSpec((_H, _H), lambda i: (0, 0)),
    ],
    out_specs=pl.BlockSpec((BLK, _H), lambda i: (i, 0)),
    out_shape=jax.ShapeDtypeStruct((NP, _H), jnp.float32),
)

_mm3 = pl.pallas_call(
    _mm3_body,
    grid=(_GRID,),
    in_specs=[
        pl.BlockSpec((NC, BLK, _H), lambda i: (0, i, 0)),
        pl.BlockSpec((BLK, _H), lambda i: (i, 0)),
        pl.BlockSpec((NC, BLK), lambda i: (0, i)),
        pl.BlockSpec((1, _H), lambda i: (0, 0)),
        pl.BlockSpec((_H, 1), lambda i: (0, 0)),
        pl.BlockSpec((BLK, 1), lambda i: (i, 0)),
        pl.BlockSpec((1, 1), lambda i: (0, 0)),
    ],
    out_specs=pl.BlockSpec((1, _G), lambda i: (0, 0)),
    out_shape=jax.ShapeDtypeStruct((1, _G), jnp.float32),
    scratch_shapes=[
        pltpu.VMEM((1, _G), jnp.float32),
        pltpu.VMEM((1, _G), jnp.float32),
    ],
)


def kernel(x, edge_index, batch, W1, b1, W2, b2, fcW, fcb):
    x_pad = jnp.pad(x, ((0, NP - _N), (0, 0)))
    pad_ids = _N + jnp.arange(EP - _E, dtype=jnp.int32) % (NP - _N)
    epad = jnp.stack([pad_ids, pad_ids])
    eidx = jnp.concatenate([edge_index, epad], axis=1)
    row_r = eidx[0].reshape(NW, NCH, K)
    col_r = eidx[1].reshape(NW, NCH, K)
    batch2d = jnp.pad(batch, (0, NP - _N)).reshape(NP, 1)
    zeros_nh = jnp.zeros((NP, _H), jnp.float32)
    b1r = b1.reshape(1, _H)
    b2r = b2.reshape(1, _H)

    deg2 = _deg_kernel(col_r)
    y1 = _mm1(x_pad, W1, deg2)
    acc1 = _agg_kernel(y1, row_r, col_r, zeros_nh)
    y2 = _mm2(acc1, y1, deg2, b1r, W2)
    acc2 = _agg_kernel(y2, row_r, col_r, zeros_nh)
    pooled = _mm3(acc2, y2, deg2, b2r, fcW, batch2d, fcb.reshape(1, 1))
    return pooled.reshape(_G, 1)


# speedup vs baseline: 1.1128x; 1.0384x over previous
import functools

import jax
import jax.numpy as jnp
from jax import lax
from jax.experimental import pallas as pl
from jax.experimental.pallas import tpu as pltpu
from jax.experimental.pallas import tpu_sc as plsc

NC = 2
NS = 16
L = 16
NW = NC * NS

_N = 10000
_E = 320000
_D = 128
_H = 64
_G = 512

NP = 10240
K = 128
NCH = 80
EP = NW * NCH * K
NBUF = 5
GLEAD = 3
RNDS = NCH // NBUF
RPT = NP // NS
GS = _G // NS

_mesh = plsc.VectorSubcoreMesh(
    core_axis_name="c", subcore_axis_name="s", num_cores=NC, num_subcores=NS
)

_SC_PARAMS = pltpu.CompilerParams(
    needs_layout_passes=False, use_tc_tiling_on_sc=False
)

@functools.partial(
    pl.kernel,
    out_type=jax.ShapeDtypeStruct((NC, NP), jnp.float32),
    mesh=_mesh,
    compiler_params=_SC_PARAMS,
    scratch_types=[
        pltpu.VMEM((NCH, K), jnp.int32),
        pltpu.VMEM((NP,), jnp.float32),
        pltpu.VMEM((NS, RPT), jnp.float32),
        pltpu.VMEM((RPT,), jnp.float32),
        pltpu.VMEM_SHARED((NS, NP), jnp.float32),
    ],
)
def _deg_kernel(col_hbm, deg_out, colv, degp, comb, outv, dsh):
    c = lax.axis_index("c")
    sid = lax.axis_index("s")
    wid = c * NS + sid

    zero16 = jnp.zeros((L,), jnp.float32)

    @pl.loop(0, NP // L)
    def _(i):
        degp[pl.ds(i * L, L)] = zero16

    pltpu.sync_copy(col_hbm.at[wid], colv)
    ones = jnp.ones((L,), jnp.float32)

    @pl.loop(0, NCH)
    def _(j):
        for t in range(K // L):
            idx = colv[j, pl.ds(t * L, L)]
            plsc.addupdate_scatter(degp, [idx], ones)

    pltpu.sync_copy(degp, dsh.at[sid])
    plsc.subcore_barrier()
    pltpu.sync_copy(dsh.at[:, pl.ds(sid * RPT, RPT)], comb)

    @pl.loop(0, RPT // L)
    def _(t):
        a = comb[0, pl.ds(t * L, L)]
        for r in range(1, NS):
            a = a + comb[r, pl.ds(t * L, L)]
        outv[pl.ds(t * L, L)] = a

    pltpu.sync_copy(outv, deg_out.at[c, pl.ds(sid * RPT, RPT)])


@functools.partial(
    pl.kernel,
    out_type=jax.ShapeDtypeStruct((NC, NP, _H), jnp.float32),
    mesh=_mesh,
    compiler_params=_SC_PARAMS,
    scratch_types=[
        pltpu.VMEM((NCH, K), jnp.int32),
        pltpu.VMEM((NCH, K), jnp.int32),
        pltpu.VMEM((NBUF, K, _H), jnp.float32),
        pltpu.SemaphoreType.DMA((NBUF,)),
        pltpu.SemaphoreType.DMA((NBUF,)),
        pltpu.VMEM_SHARED((NP, _H), jnp.float32),
    ],
)
def _agg_kernel(y_hbm, row_hbm, col_hbm, zero_hbm, acc_out, rowv, colv, msg,
                semg, sems, acc_sh):
    c = lax.axis_index("c")
    sid = lax.axis_index("s")
    wid = c * NS + sid

    pltpu.sync_copy(zero_hbm.at[pl.ds(sid * RPT, RPT)], acc_sh.at[pl.ds(sid * RPT, RPT)])
    pltpu.sync_copy(row_hbm.at[wid], rowv)
    pltpu.sync_copy(col_hbm.at[wid], colv)
    plsc.subcore_barrier()

    for b0 in range(GLEAD):
        pltpu.async_copy(y_hbm.at[rowv.at[b0]], msg.at[b0], semg.at[b0])

    @pl.loop(0, RNDS)
    def _(g):
        for b in range(NBUF):
            j = g * NBUF + b
            bg = (b + GLEAD) % NBUF

            def _pref():
                def _free():
                    pltpu.make_async_copy(
                        msg.at[bg], acc_sh.at[colv.at[j - (NBUF - GLEAD)]],
                        sems.at[bg]
                    ).wait()

                if b >= NBUF - GLEAD:
                    _free()
                else:
                    pl.when(g > 0)(_free)
                pltpu.async_copy(y_hbm.at[rowv.at[j + GLEAD]], msg.at[bg], semg.at[bg])

            if b >= NBUF - GLEAD:
                pl.when(g < RNDS - 1)(_pref)
            else:
                _pref()

            pltpu.make_async_copy(y_hbm.at[rowv.at[j]], msg.at[b], semg.at[b]).wait()
            pltpu.async_copy(msg.at[b], acc_sh.at[colv.at[j]], sems.at[b], add=True)

    for i in range(NBUF):
        j = NCH - NBUF + i
        pltpu.make_async_copy(
            msg.at[j % NBUF], acc_sh.at[colv.at[j]], sems.at[j % NBUF]
        ).wait()

    plsc.subcore_barrier()
    pltpu.sync_copy(
        acc_sh.at[pl.ds(sid * RPT, RPT)], acc_out.at[c, pl.ds(sid * RPT, RPT)]
    )


BLK = 1024


def _dinv_of(deg_ref):
    deg = deg_ref[0, :] + deg_ref[1, :] + 1.0
    return 1.0 / jnp.sqrt(deg)


def _mm1_body(x_ref, w_ref, deg_ref, y_ref):
    dinv = _dinv_of(deg_ref)
    y_ref[...] = (
        jnp.dot(x_ref[...], w_ref[...], preferred_element_type=jnp.float32,
                precision=lax.Precision.HIGHEST)
        * dinv[:, None]
    )


def _mm2_body(acc_ref, y1_ref, deg_ref, b1_ref, w2_ref, y2_ref):
    dinv = _dinv_of(deg_ref)
    a = (acc_ref[0] + acc_ref[1] + y1_ref[...]) * dinv[:, None] + b1_ref[...]
    h = jnp.maximum(a, 0.0)
    y2_ref[...] = (
        jnp.dot(h, w2_ref[...], preferred_element_type=jnp.float32,
                precision=lax.Precision.HIGHEST) * dinv[:, None]
    )


def _mm3_body(acc_ref, y2_ref, deg_ref, b2_ref, fcw_ref, batch_ref, fcb_ref,
              out_ref, sums_ref, cnt_ref):
    i = pl.program_id(0)
    dinv = _dinv_of(deg_ref)
    h = (acc_ref[0] + acc_ref[1] + y2_ref[...]) * dinv[:, None] + b2_ref[...]
    s = jnp.dot(h, fcw_ref[...], preferred_element_type=jnp.float32,
                precision=lax.Precision.HIGHEST)
    rowid = i * BLK + lax.broadcasted_iota(jnp.int32, (BLK, 1), 0)
    gids = lax.broadcasted_iota(jnp.int32, (1, _G), 1)
    onehot = jnp.where((batch_ref[...] == gids) & (rowid < _N), 1.0, 0.0)
    ps = jnp.sum(onehot * s, axis=0)[None, :]
    pc = jnp.sum(onehot, axis=0)[None, :]

    @pl.when(i == 0)
    def _():
        sums_ref[...] = ps
        cnt_ref[...] = pc

    @pl.when(i > 0)
    def _():
        sums_ref[...] += ps
        cnt_ref[...] += pc

    out_ref[...] = sums_ref[...] / jnp.maximum(cnt_ref[...], 1.0) + fcb_ref[...]


_GRID = NP // BLK

_mm1 = pl.pallas_call(
    _mm1_body,
    grid=(_GRID,),
    in_specs=[
        pl.BlockSpec((BLK, _D), lambda i: (i, 0)),
        pl.BlockSpec((_D, _H), lambda i: (0, 0)),
        pl.BlockSpec((NC, BLK), lambda i: (0, i)),
    ],
    out_specs=pl.BlockSpec((BLK, _H), lambda i: (i, 0)),
    out_shape=jax.ShapeDtypeStruct((NP, _H), jnp.float32),
)

_mm2 = pl.pallas_call(
    _mm2_body,
    grid=(_GRID,),
    in_specs=[
        pl.BlockSpec((NC, BLK, _H), lambda i: (0, i, 0)),
        pl.BlockSpec((BLK, _H), lambda i: (i, 0)),
        pl.BlockSpec((NC, BLK), lambda i: (0, i)),
        pl.BlockSpec((1, _H), lambda i: (0, 0)),
        pl.BlockSpec((_H, _H), lambda i: (0, 0)),
    ],
    out_specs=pl.BlockSpec((BLK, _H), lambda i: (i, 0)),
    out_shape=jax.ShapeDtypeStruct((NP, _H), jnp.float32),
)

_mm3 = pl.pallas_call(
    _mm3_body,
    grid=(_GRID,),
    in_specs=[
        pl.BlockSpec((NC, BLK, _H), lambda i: (0, i, 0)),
        pl.BlockSpec((BLK, _H), lambda i: (i, 0)),
        pl.BlockSpec((NC, BLK), lambda i: (0, i)),
        pl.BlockSpec((1, _H), lambda i: (0, 0)),
        pl.BlockSpec((_H, 1), lambda i: (0, 0)),
        pl.BlockSpec((BLK, 1), lambda i: (i, 0)),
        pl.BlockSpec((1, 1), lambda i: (0, 0)),
    ],
    out_specs=pl.BlockSpec((1, _G), lambda i: (0, 0)),
    out_shape=jax.ShapeDtypeStruct((1, _G), jnp.float32),
    scratch_shapes=[
        pltpu.VMEM((1, _G), jnp.float32),
        pltpu.VMEM((1, _G), jnp.float32),
    ],
)


def kernel(x, edge_index, batch, W1, b1, W2, b2, fcW, fcb):
    x_pad = jnp.pad(x, ((0, NP - _N), (0, 0)))
    pad_ids = _N + jnp.arange(EP - _E, dtype=jnp.int32) % (NP - _N)
    epad = jnp.stack([pad_ids, pad_ids])
    eidx = jnp.concatenate([edge_index, epad], axis=1)
    row_r = eidx[0].reshape(NW, NCH, K)
    col_r = eidx[1].reshape(NW, NCH, K)
    batch2d = jnp.pad(batch, (0, NP - _N)).reshape(NP, 1)
    zeros_nh = jnp.zeros((NP, _H), jnp.float32)
    b1r = b1.reshape(1, _H)
    b2r = b2.reshape(1, _H)

    deg2 = _deg_kernel(col_r)
    y1 = _mm1(x_pad, W1, deg2)
    acc1 = _agg_kernel(y1, row_r, col_r, zeros_nh)
    y2 = _mm2(acc1, y1, deg2, b1r, W2)
    acc2 = _agg_kernel(y2, row_r, col_r, zeros_nh)
    pooled = _mm3(acc2, y2, deg2, b2r, fcW, batch2d, fcb.reshape(1, 1))
    return pooled.reshape(_G, 1)


# speedup vs baseline: 1.1244x; 1.0104x over previous
import functools

import jax
import jax.numpy as jnp
from jax import lax
from jax.experimental import pallas as pl
from jax.experimental.pallas import tpu as pltpu
from jax.experimental.pallas import tpu_sc as plsc

NC = 2
NS = 16
L = 16
NW = NC * NS

_N = 10000
_E = 320000
_D = 128
_H = 64
_G = 512

NP = 10240
K = 128
NCH = 80
EP = NW * NCH * K
NBUF = 8
GLEAD = 6
RNDS = NCH // NBUF
RPT = NP // NS
GS = _G // NS

_mesh = plsc.VectorSubcoreMesh(
    core_axis_name="c", subcore_axis_name="s", num_cores=NC, num_subcores=NS
)

_SC_PARAMS = pltpu.CompilerParams(
    needs_layout_passes=False, use_tc_tiling_on_sc=False
)

@functools.partial(
    pl.kernel,
    out_type=jax.ShapeDtypeStruct((NC, NP), jnp.float32),
    mesh=_mesh,
    compiler_params=_SC_PARAMS,
    scratch_types=[
        pltpu.VMEM((NCH, K), jnp.int32),
        pltpu.VMEM((NP,), jnp.float32),
        pltpu.VMEM((NS, RPT), jnp.float32),
        pltpu.VMEM((RPT,), jnp.float32),
        pltpu.VMEM_SHARED((NS, NP), jnp.float32),
    ],
)
def _deg_kernel(col_hbm, deg_out, colv, degp, comb, outv, dsh):
    c = lax.axis_index("c")
    sid = lax.axis_index("s")
    wid = c * NS + sid

    zero16 = jnp.zeros((L,), jnp.float32)

    @pl.loop(0, NP // L)
    def _(i):
        degp[pl.ds(i * L, L)] = zero16

    pltpu.sync_copy(col_hbm.at[wid], colv)
    ones = jnp.ones((L,), jnp.float32)

    @pl.loop(0, NCH)
    def _(j):
        for t in range(K // L):
            idx = colv[j, pl.ds(t * L, L)]
            plsc.addupdate_scatter(degp, [idx], ones)

    pltpu.sync_copy(degp, dsh.at[sid])
    plsc.subcore_barrier()
    pltpu.sync_copy(dsh.at[:, pl.ds(sid * RPT, RPT)], comb)

    @pl.loop(0, RPT // L)
    def _(t):
        a = comb[0, pl.ds(t * L, L)]
        for r in range(1, NS):
            a = a + comb[r, pl.ds(t * L, L)]
        outv[pl.ds(t * L, L)] = a

    pltpu.sync_copy(outv, deg_out.at[c, pl.ds(sid * RPT, RPT)])


@functools.partial(
    pl.kernel,
    out_type=jax.ShapeDtypeStruct((NC, NP, _H), jnp.float32),
    mesh=_mesh,
    compiler_params=_SC_PARAMS,
    scratch_types=[
        pltpu.VMEM((NCH, K), jnp.int32),
        pltpu.VMEM((NCH, K), jnp.int32),
        pltpu.VMEM((NBUF, K, _H), jnp.float32),
        pltpu.SemaphoreType.DMA((NBUF,)),
        pltpu.SemaphoreType.DMA((NBUF,)),
        pltpu.VMEM_SHARED((NP, _H), jnp.float32),
    ],
)
def _agg_kernel(y_hbm, row_hbm, col_hbm, zero_hbm, acc_out, rowv, colv, msg,
                semg, sems, acc_sh):
    c = lax.axis_index("c")
    sid = lax.axis_index("s")
    wid = c * NS + sid

    pltpu.sync_copy(zero_hbm.at[pl.ds(sid * RPT, RPT)], acc_sh.at[pl.ds(sid * RPT, RPT)])
    pltpu.sync_copy(row_hbm.at[wid], rowv)
    pltpu.sync_copy(col_hbm.at[wid], colv)
    plsc.subcore_barrier()

    for b0 in range(GLEAD):
        pltpu.async_copy(y_hbm.at[rowv.at[b0]], msg.at[b0], semg.at[b0])

    @pl.loop(0, RNDS)
    def _(g):
        for b in range(NBUF):
            j = g * NBUF + b
            bg = (b + GLEAD) % NBUF

            def _pref():
                def _free():
                    pltpu.make_async_copy(
                        msg.at[bg], acc_sh.at[colv.at[j - (NBUF - GLEAD)]],
                        sems.at[bg]
                    ).wait()

                if b >= NBUF - GLEAD:
                    _free()
                else:
                    pl.when(g > 0)(_free)
                pltpu.async_copy(y_hbm.at[rowv.at[j + GLEAD]], msg.at[bg], semg.at[bg])

            if b >= NBUF - GLEAD:
                pl.when(g < RNDS - 1)(_pref)
            else:
                _pref()

            pltpu.make_async_copy(y_hbm.at[rowv.at[j]], msg.at[b], semg.at[b]).wait()
            pltpu.async_copy(msg.at[b], acc_sh.at[colv.at[j]], sems.at[b], add=True)

    for i in range(NBUF):
        j = NCH - NBUF + i
        pltpu.make_async_copy(
            msg.at[j % NBUF], acc_sh.at[colv.at[j]], sems.at[j % NBUF]
        ).wait()

    plsc.subcore_barrier()
    pltpu.sync_copy(
        acc_sh.at[pl.ds(sid * RPT, RPT)], acc_out.at[c, pl.ds(sid * RPT, RPT)]
    )


BLK = 1024


def _dinv_of(deg_ref):
    deg = deg_ref[0, :] + deg_ref[1, :] + 1.0
    return 1.0 / jnp.sqrt(deg)


def _mm1_body(x_ref, w_ref, deg_ref, y_ref):
    dinv = _dinv_of(deg_ref)
    y_ref[...] = (
        jnp.dot(x_ref[...], w_ref[...], preferred_element_type=jnp.float32,
                precision=lax.Precision.HIGHEST)
        * dinv[:, None]
    )


def _mm2_body(acc_ref, y1_ref, deg_ref, b1_ref, w2_ref, y2_ref):
    dinv = _dinv_of(deg_ref)
    a = (acc_ref[0] + acc_ref[1] + y1_ref[...]) * dinv[:, None] + b1_ref[...]
    h = jnp.maximum(a, 0.0)
    y2_ref[...] = (
        jnp.dot(h, w2_ref[...], preferred_element_type=jnp.float32,
                precision=lax.Precision.HIGHEST) * dinv[:, None]
    )


def _mm3_body(acc_ref, y2_ref, deg_ref, b2_ref, fcw_ref, batch_ref, fcb_ref,
              out_ref, sums_ref, cnt_ref):
    i = pl.program_id(0)
    dinv = _dinv_of(deg_ref)
    h = (acc_ref[0] + acc_ref[1] + y2_ref[...]) * dinv[:, None] + b2_ref[...]
    s = jnp.dot(h, fcw_ref[...], preferred_element_type=jnp.float32,
                precision=lax.Precision.HIGHEST)
    rowid = i * BLK + lax.broadcasted_iota(jnp.int32, (BLK, 1), 0)
    gids = lax.broadcasted_iota(jnp.int32, (1, _G), 1)
    onehot = jnp.where((batch_ref[...] == gids) & (rowid < _N), 1.0, 0.0)
    ps = jnp.sum(onehot * s, axis=0)[None, :]
    pc = jnp.sum(onehot, axis=0)[None, :]

    @pl.when(i == 0)
    def _():
        sums_ref[...] = ps
        cnt_ref[...] = pc

    @pl.when(i > 0)
    def _():
        sums_ref[...] += ps
        cnt_ref[...] += pc

    out_ref[...] = sums_ref[...] / jnp.maximum(cnt_ref[...], 1.0) + fcb_ref[...]


_GRID = NP // BLK

_mm1 = pl.pallas_call(
    _mm1_body,
    grid=(_GRID,),
    in_specs=[
        pl.BlockSpec((BLK, _D), lambda i: (i, 0)),
        pl.BlockSpec((_D, _H), lambda i: (0, 0)),
        pl.BlockSpec((NC, BLK), lambda i: (0, i)),
    ],
    out_specs=pl.BlockSpec((BLK, _H), lambda i: (i, 0)),
    out_shape=jax.ShapeDtypeStruct((NP, _H), jnp.float32),
)

_mm2 = pl.pallas_call(
    _mm2_body,
    grid=(_GRID,),
    in_specs=[
        pl.BlockSpec((NC, BLK, _H), lambda i: (0, i, 0)),
        pl.BlockSpec((BLK, _H), lambda i: (i, 0)),
        pl.BlockSpec((NC, BLK), lambda i: (0, i)),
        pl.BlockSpec((1, _H), lambda i: (0, 0)),
        pl.BlockSpec((_H, _H), lambda i: (0, 0)),
    ],
    out_specs=pl.BlockSpec((BLK, _H), lambda i: (i, 0)),
    out_shape=jax.ShapeDtypeStruct((NP, _H), jnp.float32),
)

_mm3 = pl.pallas_call(
    _mm3_body,
    grid=(_GRID,),
    in_specs=[
        pl.BlockSpec((NC, BLK, _H), lambda i: (0, i, 0)),
        pl.BlockSpec((BLK, _H), lambda i: (i, 0)),
        pl.BlockSpec((NC, BLK), lambda i: (0, i)),
        pl.BlockSpec((1, _H), lambda i: (0, 0)),
        pl.BlockSpec((_H, 1), lambda i: (0, 0)),
        pl.BlockSpec((BLK, 1), lambda i: (i, 0)),
        pl.BlockSpec((1, 1), lambda i: (0, 0)),
    ],
    out_specs=pl.BlockSpec((1, _G), lambda i: (0, 0)),
    out_shape=jax.ShapeDtypeStruct((1, _G), jnp.float32),
    scratch_shapes=[
        pltpu.VMEM((1, _G), jnp.float32),
        pltpu.VMEM((1, _G), jnp.float32),
    ],
)


def kernel(x, edge_index, batch, W1, b1, W2, b2, fcW, fcb):
    x_pad = jnp.pad(x, ((0, NP - _N), (0, 0)))
    pad_ids = _N + jnp.arange(EP - _E, dtype=jnp.int32) % (NP - _N)
    epad = jnp.stack([pad_ids, pad_ids])
    eidx = jnp.concatenate([edge_index, epad], axis=1)
    row_r = eidx[0].reshape(NW, NCH, K)
    col_r = eidx[1].reshape(NW, NCH, K)
    batch2d = jnp.pad(batch, (0, NP - _N)).reshape(NP, 1)
    zeros_nh = jnp.zeros((NP, _H), jnp.float32)
    b1r = b1.reshape(1, _H)
    b2r = b2.reshape(1, _H)

    deg2 = _deg_kernel(col_r)
    y1 = _mm1(x_pad, W1, deg2)
    acc1 = _agg_kernel(y1, row_r, col_r, zeros_nh)
    y2 = _mm2(acc1, y1, deg2, b1r, W2)
    acc2 = _agg_kernel(y2, row_r, col_r, zeros_nh)
    pooled = _mm3(acc2, y2, deg2, b2r, fcW, batch2d, fcb.reshape(1, 1))
    return pooled.reshape(_G, 1)


# speedup vs baseline: 1.1618x; 1.0333x over previous
import functools

import jax
import jax.numpy as jnp
from jax import lax
from jax.experimental import pallas as pl
from jax.experimental.pallas import tpu as pltpu
from jax.experimental.pallas import tpu_sc as plsc

NC = 2
NS = 16
L = 16
NW = NC * NS

_N = 10000
_E = 320000
_D = 128
_H = 64
_G = 512

NP = 10240
K = 128
NCH = 80
EP = NW * NCH * K
NBUF = 8
GLEAD = 6
RNDS = NCH // NBUF
RPT = NP // NS
GS = _G // NS

_mesh = plsc.VectorSubcoreMesh(
    core_axis_name="c", subcore_axis_name="s", num_cores=NC, num_subcores=NS
)

_SC_PARAMS = pltpu.CompilerParams(
    needs_layout_passes=False, use_tc_tiling_on_sc=False
)

@functools.partial(
    pl.kernel,
    out_type=jax.ShapeDtypeStruct((NC, NP), jnp.float32),
    mesh=_mesh,
    compiler_params=_SC_PARAMS,
    scratch_types=[
        pltpu.VMEM((NCH, K), jnp.int32),
        pltpu.VMEM((NP,), jnp.float32),
        pltpu.VMEM((NS, RPT), jnp.float32),
        pltpu.VMEM((RPT,), jnp.float32),
        pltpu.VMEM_SHARED((NS, NP), jnp.float32),
    ],
)
def _deg_kernel(col_hbm, deg_out, colv, degp, comb, outv, dsh):
    c = lax.axis_index("c")
    sid = lax.axis_index("s")
    wid = c * NS + sid

    zero16 = jnp.zeros((L,), jnp.float32)

    @pl.loop(0, NP // L)
    def _(i):
        degp[pl.ds(i * L, L)] = zero16

    pltpu.sync_copy(col_hbm.at[wid], colv)
    ones = jnp.ones((L,), jnp.float32)

    @pl.loop(0, NCH)
    def _(j):
        for t in range(K // L):
            idx = colv[j, pl.ds(t * L, L)]
            plsc.addupdate_scatter(degp, [idx], ones)

    pltpu.sync_copy(degp, dsh.at[sid])
    plsc.subcore_barrier()
    pltpu.sync_copy(dsh.at[:, pl.ds(sid * RPT, RPT)], comb)

    @pl.loop(0, RPT // L)
    def _(t):
        a = comb[0, pl.ds(t * L, L)]
        for r in range(1, NS):
            a = a + comb[r, pl.ds(t * L, L)]
        outv[pl.ds(t * L, L)] = a

    pltpu.sync_copy(outv, deg_out.at[c, pl.ds(sid * RPT, RPT)])


@functools.partial(
    pl.kernel,
    out_type=jax.ShapeDtypeStruct((NC, NP, _H), jnp.float32),
    mesh=_mesh,
    compiler_params=_SC_PARAMS,
    scratch_types=[
        pltpu.VMEM((NCH, K), jnp.int32),
        pltpu.VMEM((NCH, K), jnp.int32),
        pltpu.VMEM((NBUF, K, _H), jnp.float32),
        pltpu.SemaphoreType.DMA((NBUF,)),
        pltpu.SemaphoreType.DMA((NBUF,)),
        pltpu.VMEM_SHARED((NP, _H), jnp.float32),
    ],
)
def _agg_kernel(y_hbm, row_hbm, col_hbm, zero_hbm, acc_out, rowv, colv, msg,
                semg, sems, acc_sh):
    c = lax.axis_index("c")
    sid = lax.axis_index("s")
    wid = c * NS + sid

    pltpu.sync_copy(zero_hbm.at[pl.ds(sid * RPT, RPT)], acc_sh.at[pl.ds(sid * RPT, RPT)])
    pltpu.sync_copy(row_hbm.at[wid], rowv)
    pltpu.sync_copy(col_hbm.at[wid], colv)
    plsc.subcore_barrier()

    for b0 in range(GLEAD):
        pltpu.async_copy(y_hbm.at[rowv.at[b0]], msg.at[b0], semg.at[b0])

    @pl.loop(0, RNDS)
    def _(g):
        for b in range(NBUF):
            j = g * NBUF + b
            bg = (b + GLEAD) % NBUF

            def _pref():
                def _free():
                    pltpu.make_async_copy(
                        msg.at[bg], acc_sh.at[colv.at[j - (NBUF - GLEAD)]],
                        sems.at[bg]
                    ).wait()

                if b >= NBUF - GLEAD:
                    _free()
                else:
                    pl.when(g > 0)(_free)
                pltpu.async_copy(y_hbm.at[rowv.at[j + GLEAD]], msg.at[bg], semg.at[bg])

            if b >= NBUF - GLEAD:
                pl.when(g < RNDS - 1)(_pref)
            else:
                _pref()

            pltpu.make_async_copy(y_hbm.at[rowv.at[j]], msg.at[b], semg.at[b]).wait()
            pltpu.async_copy(msg.at[b], acc_sh.at[colv.at[j]], sems.at[b], add=True)

    for i in range(NBUF):
        j = NCH - NBUF + i
        pltpu.make_async_copy(
            msg.at[j % NBUF], acc_sh.at[colv.at[j]], sems.at[j % NBUF]
        ).wait()

    plsc.subcore_barrier()
    pltpu.sync_copy(
        acc_sh.at[pl.ds(sid * RPT, RPT)], acc_out.at[c, pl.ds(sid * RPT, RPT)]
    )


BLK = 2048


def _dinv_of(deg_ref):
    deg = deg_ref[0, :] + deg_ref[1, :] + 1.0
    return 1.0 / jnp.sqrt(deg)


def _mm1_body(x_ref, w_ref, deg_ref, y_ref):
    dinv = _dinv_of(deg_ref)
    y_ref[...] = (
        jnp.dot(x_ref[...], w_ref[...], preferred_element_type=jnp.float32,
                precision=lax.Precision.HIGHEST)
        * dinv[:, None]
    )


def _mm2_body(acc_ref, y1_ref, deg_ref, b1_ref, w2_ref, y2_ref):
    dinv = _dinv_of(deg_ref)
    a = (acc_ref[0] + acc_ref[1] + y1_ref[...]) * dinv[:, None] + b1_ref[...]
    h = jnp.maximum(a, 0.0)
    y2_ref[...] = (
        jnp.dot(h, w2_ref[...], preferred_element_type=jnp.float32,
                precision=lax.Precision.HIGHEST) * dinv[:, None]
    )


def _mm3_body(acc_ref, y2_ref, deg_ref, b2_ref, fcw_ref, batch_ref, fcb_ref,
              out_ref, sums_ref, cnt_ref):
    i = pl.program_id(0)
    dinv = _dinv_of(deg_ref)
    h = (acc_ref[0] + acc_ref[1] + y2_ref[...]) * dinv[:, None] + b2_ref[...]
    s = jnp.dot(h, fcw_ref[...], preferred_element_type=jnp.float32,
                precision=lax.Precision.HIGHEST)
    rowid = i * BLK + lax.broadcasted_iota(jnp.int32, (BLK, 1), 0)
    gids = lax.broadcasted_iota(jnp.int32, (1, _G), 1)
    onehot = jnp.where((batch_ref[...] == gids) & (rowid < _N), 1.0, 0.0)
    ps = jnp.sum(onehot * s, axis=0)[None, :]
    pc = jnp.sum(onehot, axis=0)[None, :]

    @pl.when(i == 0)
    def _():
        sums_ref[...] = ps
        cnt_ref[...] = pc

    @pl.when(i > 0)
    def _():
        sums_ref[...] += ps
        cnt_ref[...] += pc

    out_ref[...] = sums_ref[...] / jnp.maximum(cnt_ref[...], 1.0) + fcb_ref[...]


_GRID = NP // BLK

_mm1 = pl.pallas_call(
    _mm1_body,
    grid=(_GRID,),
    in_specs=[
        pl.BlockSpec((BLK, _D), lambda i: (i, 0)),
        pl.BlockSpec((_D, _H), lambda i: (0, 0)),
        pl.BlockSpec((NC, BLK), lambda i: (0, i)),
    ],
    out_specs=pl.BlockSpec((BLK, _H), lambda i: (i, 0)),
    out_shape=jax.ShapeDtypeStruct((NP, _H), jnp.float32),
)

_mm2 = pl.pallas_call(
    _mm2_body,
    grid=(_GRID,),
    in_specs=[
        pl.BlockSpec((NC, BLK, _H), lambda i: (0, i, 0)),
        pl.BlockSpec((BLK, _H), lambda i: (i, 0)),
        pl.BlockSpec((NC, BLK), lambda i: (0, i)),
        pl.BlockSpec((1, _H), lambda i: (0, 0)),
        pl.BlockSpec((_H, _H), lambda i: (0, 0)),
    ],
    out_specs=pl.BlockSpec((BLK, _H), lambda i: (i, 0)),
    out_shape=jax.ShapeDtypeStruct((NP, _H), jnp.float32),
)

_mm3 = pl.pallas_call(
    _mm3_body,
    grid=(_GRID,),
    in_specs=[
        pl.BlockSpec((NC, BLK, _H), lambda i: (0, i, 0)),
        pl.BlockSpec((BLK, _H), lambda i: (i, 0)),
        pl.BlockSpec((NC, BLK), lambda i: (0, i)),
        pl.BlockSpec((1, _H), lambda i: (0, 0)),
        pl.BlockSpec((_H, 1), lambda i: (0, 0)),
        pl.BlockSpec((BLK, 1), lambda i: (i, 0)),
        pl.BlockSpec((1, 1), lambda i: (0, 0)),
    ],
    out_specs=pl.BlockSpec((1, _G), lambda i: (0, 0)),
    out_shape=jax.ShapeDtypeStruct((1, _G), jnp.float32),
    scratch_shapes=[
        pltpu.VMEM((1, _G), jnp.float32),
        pltpu.VMEM((1, _G), jnp.float32),
    ],
)


def kernel(x, edge_index, batch, W1, b1, W2, b2, fcW, fcb):
    x_pad = jnp.pad(x, ((0, NP - _N), (0, 0)))
    pad_ids = _N + jnp.arange(EP - _E, dtype=jnp.int32) % (NP - _N)
    epad = jnp.stack([pad_ids, pad_ids])
    eidx = jnp.concatenate([edge_index, epad], axis=1)
    row_r = eidx[0].reshape(NW, NCH, K)
    col_r = eidx[1].reshape(NW, NCH, K)
    batch2d = jnp.pad(batch, (0, NP - _N)).reshape(NP, 1)
    zeros_nh = jnp.zeros((NP, _H), jnp.float32)
    b1r = b1.reshape(1, _H)
    b2r = b2.reshape(1, _H)

    deg2 = _deg_kernel(col_r)
    y1 = _mm1(x_pad, W1, deg2)
    acc1 = _agg_kernel(y1, row_r, col_r, zeros_nh)
    y2 = _mm2(acc1, y1, deg2, b1r, W2)
    acc2 = _agg_kernel(y2, row_r, col_r, zeros_nh)
    pooled = _mm3(acc2, y2, deg2, b2r, fcW, batch2d, fcb.reshape(1, 1))
    return pooled.reshape(_G, 1)


# speedup vs baseline: 1.1636x; 1.0016x over previous
import functools

import jax
import jax.numpy as jnp
from jax import lax
from jax.experimental import pallas as pl
from jax.experimental.pallas import tpu as pltpu
from jax.experimental.pallas import tpu_sc as plsc

NC = 2
NS = 16
L = 16
NW = NC * NS

_N = 10000
_E = 320000
_D = 128
_H = 64
_G = 512

NP = 10240
K = 128
NCH = 80
EP = NW * NCH * K
NBUF = 8
GLEAD = 6
RNDS = NCH // NBUF
RPT = NP // NS
GS = _G // NS

_mesh = plsc.VectorSubcoreMesh(
    core_axis_name="c", subcore_axis_name="s", num_cores=NC, num_subcores=NS
)

_SC_PARAMS = pltpu.CompilerParams(
    needs_layout_passes=False, use_tc_tiling_on_sc=False
)

@functools.partial(
    pl.kernel,
    out_type=jax.ShapeDtypeStruct((NC, NP), jnp.float32),
    mesh=_mesh,
    compiler_params=_SC_PARAMS,
    scratch_types=[
        pltpu.VMEM((NCH, K), jnp.int32),
        pltpu.VMEM((NP,), jnp.float32),
        pltpu.VMEM((NS, RPT), jnp.float32),
        pltpu.VMEM((RPT,), jnp.float32),
        pltpu.VMEM_SHARED((NS, NP), jnp.float32),
    ],
)
def _deg_kernel(col_hbm, deg_out, colv, degp, comb, outv, dsh):
    c = lax.axis_index("c")
    sid = lax.axis_index("s")
    wid = c * NS + sid

    zero16 = jnp.zeros((L,), jnp.float32)

    @pl.loop(0, NP // L)
    def _(i):
        degp[pl.ds(i * L, L)] = zero16

    pltpu.sync_copy(col_hbm.at[wid], colv)
    ones = jnp.ones((L,), jnp.float32)

    @pl.loop(0, NCH)
    def _(j):
        for t in range(K // L):
            idx = colv[j, pl.ds(t * L, L)]
            plsc.addupdate_scatter(degp, [idx], ones)

    pltpu.sync_copy(degp, dsh.at[sid])
    plsc.subcore_barrier()
    pltpu.sync_copy(dsh.at[:, pl.ds(sid * RPT, RPT)], comb)

    @pl.loop(0, RPT // L)
    def _(t):
        a = comb[0, pl.ds(t * L, L)]
        for r in range(1, NS):
            a = a + comb[r, pl.ds(t * L, L)]
        outv[pl.ds(t * L, L)] = a

    pltpu.sync_copy(outv, deg_out.at[c, pl.ds(sid * RPT, RPT)])


@functools.partial(
    pl.kernel,
    out_type=jax.ShapeDtypeStruct((NC, NP, _H), jnp.float32),
    mesh=_mesh,
    compiler_params=_SC_PARAMS,
    scratch_types=[
        pltpu.VMEM((NCH, K), jnp.int32),
        pltpu.VMEM((NCH, K), jnp.int32),
        pltpu.VMEM((NBUF, K, _H), jnp.float32),
        pltpu.SemaphoreType.DMA((NBUF,)),
        pltpu.SemaphoreType.DMA((NBUF,)),
        pltpu.VMEM_SHARED((NP, _H), jnp.float32),
    ],
)
def _agg_kernel(y_hbm, row_hbm, col_hbm, zero_hbm, acc_out, rowv, colv, msg,
                semg, sems, acc_sh):
    c = lax.axis_index("c")
    sid = lax.axis_index("s")
    wid = c * NS + sid

    pltpu.sync_copy(zero_hbm.at[pl.ds(sid * RPT, RPT)], acc_sh.at[pl.ds(sid * RPT, RPT)])
    pltpu.sync_copy(row_hbm.at[wid], rowv)
    pltpu.sync_copy(col_hbm.at[wid], colv)
    plsc.subcore_barrier()

    for b0 in range(GLEAD):
        pltpu.async_copy(y_hbm.at[rowv.at[b0]], msg.at[b0], semg.at[b0])

    @pl.loop(0, RNDS)
    def _(g):
        for b in range(NBUF):
            j = g * NBUF + b
            bg = (b + GLEAD) % NBUF

            def _pref():
                def _free():
                    pltpu.make_async_copy(
                        msg.at[bg], acc_sh.at[colv.at[j - (NBUF - GLEAD)]],
                        sems.at[bg]
                    ).wait()

                if b >= NBUF - GLEAD:
                    _free()
                else:
                    pl.when(g > 0)(_free)
                pltpu.async_copy(y_hbm.at[rowv.at[j + GLEAD]], msg.at[bg], semg.at[bg])

            if b >= NBUF - GLEAD:
                pl.when(g < RNDS - 1)(_pref)
            else:
                _pref()

            pltpu.make_async_copy(y_hbm.at[rowv.at[j]], msg.at[b], semg.at[b]).wait()
            pltpu.async_copy(msg.at[b], acc_sh.at[colv.at[j]], sems.at[b], add=True)

    for i in range(NBUF):
        j = NCH - NBUF + i
        pltpu.make_async_copy(
            msg.at[j % NBUF], acc_sh.at[colv.at[j]], sems.at[j % NBUF]
        ).wait()

    plsc.subcore_barrier()
    pltpu.sync_copy(
        acc_sh.at[pl.ds(sid * RPT, RPT)], acc_out.at[c, pl.ds(sid * RPT, RPT)]
    )


BLK = 5120


def _dinv_of(deg_ref):
    deg = deg_ref[0, :] + deg_ref[1, :] + 1.0
    return 1.0 / jnp.sqrt(deg)


def _mm1_body(x_ref, w_ref, deg_ref, y_ref):
    dinv = _dinv_of(deg_ref)
    y_ref[...] = (
        jnp.dot(x_ref[...], w_ref[...], preferred_element_type=jnp.float32,
                precision=lax.Precision.HIGHEST)
        * dinv[:, None]
    )


def _mm2_body(acc_ref, y1_ref, deg_ref, b1_ref, w2_ref, y2_ref):
    dinv = _dinv_of(deg_ref)
    a = (acc_ref[0] + acc_ref[1] + y1_ref[...]) * dinv[:, None] + b1_ref[...]
    h = jnp.maximum(a, 0.0)
    y2_ref[...] = (
        jnp.dot(h, w2_ref[...], preferred_element_type=jnp.float32,
                precision=lax.Precision.HIGHEST) * dinv[:, None]
    )


def _mm3_body(acc_ref, y2_ref, deg_ref, b2_ref, fcw_ref, batch_ref, fcb_ref,
              out_ref, sums_ref, cnt_ref):
    i = pl.program_id(0)
    dinv = _dinv_of(deg_ref)
    h = (acc_ref[0] + acc_ref[1] + y2_ref[...]) * dinv[:, None] + b2_ref[...]
    s = jnp.dot(h, fcw_ref[...], preferred_element_type=jnp.float32,
                precision=lax.Precision.HIGHEST)
    rowid = i * BLK + lax.broadcasted_iota(jnp.int32, (BLK, 1), 0)
    gids = lax.broadcasted_iota(jnp.int32, (1, _G), 1)
    onehot = jnp.where((batch_ref[...] == gids) & (rowid < _N), 1.0, 0.0)
    ps = jnp.sum(onehot * s, axis=0)[None, :]
    pc = jnp.sum(onehot, axis=0)[None, :]

    @pl.when(i == 0)
    def _():
        sums_ref[...] = ps
        cnt_ref[...] = pc

    @pl.when(i > 0)
    def _():
        sums_ref[...] += ps
        cnt_ref[...] += pc

    out_ref[...] = sums_ref[...] / jnp.maximum(cnt_ref[...], 1.0) + fcb_ref[...]


_GRID = NP // BLK

_mm1 = pl.pallas_call(
    _mm1_body,
    grid=(_GRID,),
    in_specs=[
        pl.BlockSpec((BLK, _D), lambda i: (i, 0)),
        pl.BlockSpec((_D, _H), lambda i: (0, 0)),
        pl.BlockSpec((NC, BLK), lambda i: (0, i)),
    ],
    out_specs=pl.BlockSpec((BLK, _H), lambda i: (i, 0)),
    out_shape=jax.ShapeDtypeStruct((NP, _H), jnp.float32),
)

_mm2 = pl.pallas_call(
    _mm2_body,
    grid=(_GRID,),
    in_specs=[
        pl.BlockSpec((NC, BLK, _H), lambda i: (0, i, 0)),
        pl.BlockSpec((BLK, _H), lambda i: (i, 0)),
        pl.BlockSpec((NC, BLK), lambda i: (0, i)),
        pl.BlockSpec((1, _H), lambda i: (0, 0)),
        pl.BlockSpec((_H, _H), lambda i: (0, 0)),
    ],
    out_specs=pl.BlockSpec((BLK, _H), lambda i: (i, 0)),
    out_shape=jax.ShapeDtypeStruct((NP, _H), jnp.float32),
)

_mm3 = pl.pallas_call(
    _mm3_body,
    grid=(_GRID,),
    in_specs=[
        pl.BlockSpec((NC, BLK, _H), lambda i: (0, i, 0)),
        pl.BlockSpec((BLK, _H), lambda i: (i, 0)),
        pl.BlockSpec((NC, BLK), lambda i: (0, i)),
        pl.BlockSpec((1, _H), lambda i: (0, 0)),
        pl.BlockSpec((_H, 1), lambda i: (0, 0)),
        pl.BlockSpec((BLK, 1), lambda i: (i, 0)),
        pl.BlockSpec((1, 1), lambda i: (0, 0)),
    ],
    out_specs=pl.BlockSpec((1, _G), lambda i: (0, 0)),
    out_shape=jax.ShapeDtypeStruct((1, _G), jnp.float32),
    scratch_shapes=[
        pltpu.VMEM((1, _G), jnp.float32),
        pltpu.VMEM((1, _G), jnp.float32),
    ],
)


def kernel(x, edge_index, batch, W1, b1, W2, b2, fcW, fcb):
    x_pad = jnp.pad(x, ((0, NP - _N), (0, 0)))
    pad_ids = _N + jnp.arange(EP - _E, dtype=jnp.int32) % (NP - _N)
    epad = jnp.stack([pad_ids, pad_ids])
    eidx = jnp.concatenate([edge_index, epad], axis=1)
    row_r = eidx[0].reshape(NW, NCH, K)
    col_r = eidx[1].reshape(NW, NCH, K)
    batch2d = jnp.pad(batch, (0, NP - _N)).reshape(NP, 1)
    zeros_nh = jnp.zeros((NP, _H), jnp.float32)
    b1r = b1.reshape(1, _H)
    b2r = b2.reshape(1, _H)

    deg2 = _deg_kernel(col_r)
    y1 = _mm1(x_pad, W1, deg2)
    acc1 = _agg_kernel(y1, row_r, col_r, zeros_nh)
    y2 = _mm2(acc1, y1, deg2, b1r, W2)
    acc2 = _agg_kernel(y2, row_r, col_r, zeros_nh)
    pooled = _mm3(acc2, y2, deg2, b2r, fcW, batch2d, fcb.reshape(1, 1))
    return pooled.reshape(_G, 1)
